# Initial kernel scaffold; baseline (speedup 1.0000x reference)
import jax
import jax.numpy as jnp
from jax.experimental import pallas as pl


def kernel(x, table):
    raise NotImplementedError("write your pallas kernel here")


# speedup vs baseline: 1.1036x; 1.1036x over previous
import functools

import jax
import jax.numpy as jnp
from jax import lax
from jax.experimental import pallas as pl
from jax.experimental.pallas import tpu as pltpu
from jax.experimental.pallas import tpu_sc as plsc

_NC = 2
_NS = 16
_NW = _NC * _NS

_CHUNK = 1600


@functools.lru_cache(maxsize=None)
def _emb_call(n_total: int, d: int):
    per_w = n_total // _NW
    n_chunks = per_w // _CHUNK
    assert per_w % _CHUNK == 0 and n_total % _NW == 0

    mesh = plsc.VectorSubcoreMesh(core_axis_name="c", subcore_axis_name="s")

    @functools.partial(
        pl.kernel,
        mesh=mesh,
        out_type=jax.ShapeDtypeStruct((n_total, d), jnp.float32),
        compiler_params=pltpu.CompilerParams(use_tc_tiling_on_sc=False),
        scratch_types=[
            pltpu.VMEM((_CHUNK,), jnp.int32),
            pltpu.VMEM((_CHUNK, d), jnp.float32),
            pltpu.SemaphoreType.DMA,
        ],
    )
    def k(x_hbm, table_hbm, out_hbm, idx_v, rows_v, sem):
        wid = lax.axis_index("s") * _NC + lax.axis_index("c")
        base = wid * per_w

        def body(g, carry):
            off = base + g * _CHUNK
            pltpu.sync_copy(x_hbm.at[pl.ds(off, _CHUNK)], idx_v)
            pltpu.async_copy(table_hbm.at[idx_v], rows_v, sem).wait()
            pltpu.sync_copy(rows_v, out_hbm.at[pl.ds(off, _CHUNK)])
            return carry

        lax.fori_loop(0, n_chunks, body, 0)

    return k


def kernel(x, table):
    b, s = x.shape
    d = table.shape[1]
    xf = x.reshape(b * s).astype(jnp.int32)
    out = _emb_call(b * s, d)(xf, table)
    return out.reshape(b, s, d)


# speedup vs baseline: 1.1131x; 1.0086x over previous
import functools

import jax
import jax.numpy as jnp
from jax import lax
from jax.experimental import pallas as pl
from jax.experimental.pallas import tpu as pltpu
from jax.experimental.pallas import tpu_sc as plsc

_NC = 2
_NS = 16
_NW = _NC * _NS

_CHUNK = 1600


@functools.lru_cache(maxsize=None)
def _emb_call(n_total: int, d: int):
    per_w = n_total // _NW
    n_chunks = per_w // _CHUNK
    assert per_w % _CHUNK == 0 and n_total % _NW == 0

    mesh = plsc.VectorSubcoreMesh(core_axis_name="c", subcore_axis_name="s")

    @functools.partial(
        pl.kernel,
        mesh=mesh,
        out_type=jax.ShapeDtypeStruct((n_total, d), jnp.float32),
        compiler_params=pltpu.CompilerParams(use_tc_tiling_on_sc=False),
        scratch_types=[
            pltpu.VMEM((per_w,), jnp.int32),
            pltpu.VMEM((_CHUNK, d), jnp.float32),
            pltpu.VMEM((_CHUNK, d), jnp.float32),
            pltpu.SemaphoreType.DMA,
            pltpu.SemaphoreType.DMA,
            pltpu.SemaphoreType.DMA,
            pltpu.SemaphoreType.DMA,
        ],
    )
    def k(x_hbm, table_hbm, out_hbm, idx_all, rows0, rows1, g0, g1, s0, s1):
        wid = lax.axis_index("s") * _NC + lax.axis_index("c")
        base = wid * per_w
        pltpu.sync_copy(x_hbm.at[pl.ds(base, per_w)], idx_all)

        rows = (rows0, rows1)
        gsem = (g0, g1)
        ssem = (s0, s1)

        def gather_start(g):
            b = g % 2
            return pltpu.async_copy(
                table_hbm.at[idx_all.at[pl.ds(g * _CHUNK, _CHUNK)]],
                rows[b], gsem[b])

        def store_start(g):
            b = g % 2
            return pltpu.async_copy(
                rows[b], out_hbm.at[pl.ds(base + g * _CHUNK, _CHUNK)],
                ssem[b])

        gathers = [gather_start(0)]
        stores = [None, None]
        for g in range(n_chunks):
            b = g % 2
            if g + 1 < n_chunks:
                if stores[1 - b] is not None:
                    stores[1 - b].wait()
                gathers.append(gather_start(g + 1))
            gathers[g].wait()
            stores[b] = store_start(g)
        stores[(n_chunks - 1) % 2].wait()
        if n_chunks > 1:
            stores[n_chunks % 2].wait()

    return k


def kernel(x, table):
    b, s = x.shape
    d = table.shape[1]
    xf = x.reshape(b * s).astype(jnp.int32)
    out = _emb_call(b * s, d)(xf, table)
    return out.reshape(b, s, d)
